# baseline (speedup 1.0000x reference)
import functools

import jax
import jax.numpy as jnp
from jax import lax
from jax.experimental import pallas as pl
from jax.experimental.pallas import tpu as pltpu
from jax.experimental.pallas import tpu_sc as plsc

N = 10000
H = 32
D_IN = 128
D_OUT = 128
K = 15
KP_EXTENT = 0.12

NP = 10240
EP = NP * H
NW = 32
EW = EP // NW
CHU = 128
NCH = EW // CHU
B = 256
GRID = NP // B

_MESH = dict(core_axis_name="c", subcore_axis_name="s")


def _sc_fused(feats, px, py, pz, idx_flat, kp_rep):
    mesh = plsc.VectorSubcoreMesh(**_MESH)

    @functools.partial(
        pl.kernel,
        out_type=(
            jax.ShapeDtypeStruct((EP * D_IN // 2,), jnp.int32),
            jax.ShapeDtypeStruct((16, EP), jnp.float32),
        ),
        mesh=mesh,
        compiler_params=pltpu.CompilerParams(needs_layout_passes=False),
        scratch_types=[
            pltpu.VMEM((N,), jnp.float32),
            pltpu.VMEM((N,), jnp.float32),
            pltpu.VMEM((N,), jnp.float32),
            pltpu.VMEM((EW,), jnp.int32),
            pltpu.VMEM((720,), jnp.float32),
            pltpu.VMEM((16, CHU), jnp.float32),
            pltpu.VMEM((CHU, D_IN), jnp.float32),
            pltpu.VMEM((CHU, D_IN), jnp.float32),
            pltpu.VMEM((CHU * D_IN // 2,), jnp.int32),
            pltpu.VMEM((CHU * D_IN // 2,), jnp.int32),
            pltpu.VMEM((CHU,), jnp.int32),
            pltpu.VMEM((CHU,), jnp.int32),
            pltpu.SemaphoreType.DMA,
            pltpu.SemaphoreType.DMA,
            pltpu.SemaphoreType.DMA,
            pltpu.SemaphoreType.DMA,
        ],
    )
    def fused(feat_h, px_h, py_h, pz_h, idx_h, kp_h, gx_h, w_h,
              px_v, py_v, pz_v, idx_v, kp_v, wout_v,
              rows0, rows1, r16_0, r16_1, idxc0, idxc1,
              sin0, sin1, sout0, sout1):
        wid = lax.axis_index("s") * 2 + lax.axis_index("c")
        base = wid * EW
        pltpu.sync_copy(px_h, px_v)
        pltpu.sync_copy(py_h, py_v)
        pltpu.sync_copy(pz_h, pz_v)
        pltpu.sync_copy(idx_h.at[pl.ds(base, EW)], idx_v)
        pltpu.sync_copy(kp_h, kp_v)

        kvecs = [
            (kp_v[pl.ds(k * 48, 16)],
             kp_v[pl.ds(k * 48 + 16, 16)],
             kp_v[pl.ds(k * 48 + 32, 16)])
            for k in range(K)
        ]
        lane = lax.iota(jnp.int32, 16)
        inv_ext = jnp.float32(1.0 / KP_EXTENT)
        rows = (rows0, rows1)
        rows16 = (r16_0, r16_1)
        idxc = (idxc0, idxc1)
        sin = (sin0, sin1)
        sout = (sout0, sout1)

        def weights_for(c0):
            @pl.loop(0, CHU, step=16)
            def _vec(v0):
                i_nbr = idx_v[pl.ds(c0 + v0, 16)]
                e_g = base + c0 + v0 + lane
                i_ctr = lax.shift_right_logical(e_g, 5)
                xn = plsc.load_gather(px_v, [i_nbr])
                yn = plsc.load_gather(py_v, [i_nbr])
                zn = plsc.load_gather(pz_v, [i_nbr])
                xc = plsc.load_gather(px_v, [i_ctr])
                yc = plsc.load_gather(py_v, [i_ctr])
                zc = plsc.load_gather(pz_v, [i_ctr])
                rx = xn - xc
                ry = yn - yc
                rz = zn - zc
                row = v0 + lane
                for k in range(K):
                    kx, ky, kz = kvecs[k]
                    dx = rx - kx
                    dy = ry - ky
                    dz = rz - kz
                    d2 = jnp.maximum(dx * dx + dy * dy + dz * dz,
                                     jnp.float32(1e-24))
                    bits = plsc.bitcast(d2, jnp.int32)
                    seed = jnp.int32(0x5F3759DF) - lax.shift_right_logical(
                        bits, 1)
                    r = plsc.bitcast(seed, jnp.float32)
                    for _ in range(3):
                        r = r * (jnp.float32(1.5)
                                 - jnp.float32(0.5) * d2 * r * r)
                    dist = d2 * r
                    w = jnp.maximum(jnp.float32(1.0) - dist * inv_ext,
                                    jnp.float32(0.0))
                    kfull = jnp.full((16,), k, jnp.int32)
                    plsc.store_scatter(wout_v, [kfull, row], w)

            pltpu.sync_copy(wout_v, w_h.at[:, pl.ds(base + c0, CHU)])

        @pl.loop(0, NCH, step=2)
        def _pair(g):
            for b in range(2):
                gg = g + b
                c0 = gg * CHU

                @pl.when(g >= 2)
                def _drain():
                    pltpu.make_async_copy(
                        rows16[b], gx_h.at[pl.ds(0, CHU * D_IN // 2)],
                        sout[b]).wait()

                pltpu.sync_copy(idx_h.at[pl.ds(base + c0, CHU)], idxc[b])
                in_h = pltpu.async_copy(feat_h.at[idxc[b]], rows[b],
                                        sin[b])
                weights_for(c0)
                in_h.wait()

                @pl.loop(0, CHU)
                def _cvt(rr):
                    rfull = jnp.full((16,), rr, jnp.int32)
                    for cg in range(0, D_IN, 32):
                        a = plsc.load_gather(rows[b], [rfull, cg + lane])
                        cvec = plsc.load_gather(rows[b],
                                                [rfull, cg + 16 + lane])
                        pk = plsc.pack(a, cvec,
                                       format=plsc.PackFormat.INTERLEAVED)
                        pk32 = plsc.bitcast(pk, jnp.int32)
                        plsc.store_scatter(
                            rows16[b],
                            [rr * (D_IN // 2) + cg // 2 + lane], pk32)

                pltpu.async_copy(rows16[b],
                                 gx_h.at[pl.ds((base + c0) * (D_IN // 2),
                                               CHU * D_IN // 2)],
                                 sout[b])

        for b in range(2):
            pltpu.make_async_copy(
                rows16[b], gx_h.at[pl.ds(0, CHU * D_IN // 2)],
                sout[b]).wait()

    return fused(feats, px, py, pz, idx_flat, kp_rep)


GP = 32
GE = GP * H
NG = B // GP


def _tc_body(gx_ref, wt_ref, mask_ref, w2_ref, out_ref):
    feats = gx_ref[...]
    mask = mask_ref[...]
    wfs = []
    for g in range(NG):
        wt_g = wt_ref[:, g * GE:(g + 1) * GE].astype(jnp.bfloat16)
        lhs = jnp.tile(wt_g, (GP, 1)) * mask
        x_g = feats[g * GE:(g + 1) * GE, :]
        acc = jnp.dot(lhs, x_g, preferred_element_type=jnp.float32)
        wfs.append(acc.reshape(GP, 16 * D_IN))
    wf = jnp.concatenate(wfs, axis=0).astype(jnp.bfloat16)
    out = jnp.dot(wf, w2_ref[...], preferred_element_type=jnp.float32)
    out_ref[...] = jnp.where(out > 0, out, 0.1 * out)


def _tc_compute(gx, wt, mask, w2pad):
    return pl.pallas_call(
        _tc_body,
        grid=(GRID,),
        in_specs=[
            pl.BlockSpec((B * H, D_IN), lambda i: (i, 0)),
            pl.BlockSpec((16, B * H), lambda i: (0, i)),
            pl.BlockSpec((GP * 16, GE), lambda i: (0, 0)),
            pl.BlockSpec((16 * D_IN, D_OUT), lambda i: (0, 0)),
        ],
        out_specs=pl.BlockSpec((B, D_OUT), lambda i: (i, 0)),
        out_shape=jax.ShapeDtypeStruct((NP, D_OUT), jnp.float32),
    )(gx, wt, mask, w2pad)


def kernel(points, features, neighbor_indices, kernel_points, weights):
    px = points[:, 0]
    py = points[:, 1]
    pz = points[:, 2]
    idx_pad = jnp.pad(neighbor_indices, ((0, NP - N), (0, 0)))
    idx_flat = idx_pad.reshape(EP)
    kp_rep = jnp.tile(kernel_points.reshape(K * 3, 1), (1, 16)).reshape(720)
    b16 = jnp.arange(16, dtype=jnp.int32)
    idx32 = jnp.stack([b16, b16 + 16], axis=1).reshape(32)
    perm = jnp.concatenate([idx32 + 32 * g for g in range(4)])
    w2pad = jnp.concatenate(
        [weights.reshape(K * D_IN, D_OUT),
         jnp.zeros((D_IN, D_OUT), jnp.float32)])
    w2pad = (w2pad.reshape(16, D_IN, D_OUT)[:, perm, :]
             .reshape(16 * D_IN, D_OUT).astype(jnp.bfloat16))
    mask = (jnp.arange(GP * 16)[:, None] // 16
            == jnp.arange(GE)[None, :] // H).astype(jnp.bfloat16)
    gx32, wt = _sc_fused(features, px, py, pz, idx_flat, kp_rep)
    gx = jax.lax.bitcast_convert_type(
        gx32.reshape(EP, D_IN // 2), jnp.bfloat16).reshape(EP, D_IN)
    out = _tc_compute(gx, wt, mask, w2pad)
    return out[:N]


# speedup vs baseline: 2.5304x; 2.5304x over previous
import functools

import jax
import jax.numpy as jnp
from jax import lax
from jax.experimental import pallas as pl
from jax.experimental.pallas import tpu as pltpu
from jax.experimental.pallas import tpu_sc as plsc

N = 10000
H = 32
D_IN = 128
D_OUT = 128
K = 15
KP_EXTENT = 0.12

NP = 10240
EP = NP * H
NW = 32
EW = EP // NW
CHU = 256
NCH = EW // CHU
B = 256
GRID = NP // B

_MESH = dict(core_axis_name="c", subcore_axis_name="s")


def _sc_fused(feats, px, py, pz, idx_flat, kp_rep):
    mesh = plsc.VectorSubcoreMesh(**_MESH)

    @functools.partial(
        pl.kernel,
        out_type=(
            jax.ShapeDtypeStruct((EP, D_IN), jnp.float32),
            jax.ShapeDtypeStruct((16, EP), jnp.float32),
        ),
        mesh=mesh,
        compiler_params=pltpu.CompilerParams(needs_layout_passes=False),
        scratch_types=[
            pltpu.VMEM((N,), jnp.float32),
            pltpu.VMEM((N,), jnp.float32),
            pltpu.VMEM((N,), jnp.float32),
            pltpu.VMEM((EW,), jnp.int32),
            pltpu.VMEM((720,), jnp.float32),
            pltpu.VMEM((16, CHU), jnp.float32),
            pltpu.VMEM((CHU, D_IN), jnp.float32),
            pltpu.VMEM((CHU, D_IN), jnp.float32),
            pltpu.VMEM((CHU,), jnp.int32),
            pltpu.VMEM((CHU,), jnp.int32),
            pltpu.SemaphoreType.DMA,
            pltpu.SemaphoreType.DMA,
            pltpu.SemaphoreType.DMA,
            pltpu.SemaphoreType.DMA,
        ],
    )
    def fused(feat_h, px_h, py_h, pz_h, idx_h, kp_h, gx_h, w_h,
              px_v, py_v, pz_v, idx_v, kp_v, wout_v,
              rows0, rows1, idxc0, idxc1,
              sin0, sin1, sout0, sout1):
        wid = lax.axis_index("s") * 2 + lax.axis_index("c")
        base = wid * EW
        pltpu.sync_copy(px_h, px_v)
        pltpu.sync_copy(py_h, py_v)
        pltpu.sync_copy(pz_h, pz_v)
        pltpu.sync_copy(idx_h.at[pl.ds(base, EW)], idx_v)
        pltpu.sync_copy(kp_h, kp_v)

        kvecs = [
            (kp_v[pl.ds(k * 48, 16)],
             kp_v[pl.ds(k * 48 + 16, 16)],
             kp_v[pl.ds(k * 48 + 32, 16)])
            for k in range(K)
        ]
        lane = lax.iota(jnp.int32, 16)
        inv_ext = jnp.float32(1.0 / KP_EXTENT)
        rows = (rows0, rows1)
        idxc = (idxc0, idxc1)
        sin = (sin0, sin1)
        sout = (sout0, sout1)

        def weights_for(c0):
            @pl.loop(0, CHU, step=16)
            def _vec(v0):
                i_nbr = idx_v[pl.ds(c0 + v0, 16)]
                e_g = base + c0 + v0 + lane
                i_ctr = lax.shift_right_logical(e_g, 5)
                xn = plsc.load_gather(px_v, [i_nbr])
                yn = plsc.load_gather(py_v, [i_nbr])
                zn = plsc.load_gather(pz_v, [i_nbr])
                xc = plsc.load_gather(px_v, [i_ctr])
                yc = plsc.load_gather(py_v, [i_ctr])
                zc = plsc.load_gather(pz_v, [i_ctr])
                rx = xn - xc
                ry = yn - yc
                rz = zn - zc
                row = v0 + lane
                for k in range(K):
                    kx, ky, kz = kvecs[k]
                    dx = rx - kx
                    dy = ry - ky
                    dz = rz - kz
                    d2 = jnp.maximum(dx * dx + dy * dy + dz * dz,
                                     jnp.float32(1e-24))
                    bits = plsc.bitcast(d2, jnp.int32)
                    seed = jnp.int32(0x5F3759DF) - lax.shift_right_logical(
                        bits, 1)
                    r = plsc.bitcast(seed, jnp.float32)
                    for _ in range(3):
                        r = r * (jnp.float32(1.5)
                                 - jnp.float32(0.5) * d2 * r * r)
                    dist = d2 * r
                    w = jnp.maximum(jnp.float32(1.0) - dist * inv_ext,
                                    jnp.float32(0.0))
                    kfull = jnp.full((16,), k, jnp.int32)
                    plsc.store_scatter(wout_v, [kfull, row], w)

            pltpu.sync_copy(wout_v, w_h.at[:, pl.ds(base + c0, CHU)])

        @pl.loop(0, NCH, step=2)
        def _pair(g):
            for b in range(2):
                gg = g + b
                c0 = gg * CHU

                @pl.when(g >= 2)
                def _drain():
                    pltpu.make_async_copy(
                        rows[b], gx_h.at[pl.ds(0, CHU), :],
                        sout[b]).wait()

                pltpu.sync_copy(idx_h.at[pl.ds(base + c0, CHU)], idxc[b])
                in_h = pltpu.async_copy(feat_h.at[idxc[b]], rows[b],
                                        sin[b])
                weights_for(c0)
                in_h.wait()
                pltpu.async_copy(rows[b],
                                 gx_h.at[pl.ds(base + c0, CHU), :],
                                 sout[b])

        for b in range(2):
            pltpu.make_async_copy(
                rows[b], gx_h.at[pl.ds(0, CHU), :], sout[b]).wait()

    return fused(feats, px, py, pz, idx_flat, kp_rep)


GP = 32
GE = GP * H
NG = B // GP


def _tc_body(gx_ref, wt_ref, mask_ref, w2_ref, out_ref):
    feats = gx_ref[...].astype(jnp.bfloat16)
    mask = mask_ref[...]
    wfs = []
    for g in range(NG):
        wt_g = wt_ref[:, g * GE:(g + 1) * GE].astype(jnp.bfloat16)
        lhs = jnp.tile(wt_g, (GP, 1)) * mask
        x_g = feats[g * GE:(g + 1) * GE, :]
        acc = jnp.dot(lhs, x_g, preferred_element_type=jnp.float32)
        wfs.append(acc.reshape(GP, 16 * D_IN))
    wf = jnp.concatenate(wfs, axis=0).astype(jnp.bfloat16)
    out = jnp.dot(wf, w2_ref[...], preferred_element_type=jnp.float32)
    out_ref[...] = jnp.where(out > 0, out, 0.1 * out)


def _tc_compute(gx, wt, mask, w2pad):
    return pl.pallas_call(
        _tc_body,
        grid=(GRID,),
        in_specs=[
            pl.BlockSpec((B * H, D_IN), lambda i: (i, 0)),
            pl.BlockSpec((16, B * H), lambda i: (0, i)),
            pl.BlockSpec((GP * 16, GE), lambda i: (0, 0)),
            pl.BlockSpec((16 * D_IN, D_OUT), lambda i: (0, 0)),
        ],
        out_specs=pl.BlockSpec((B, D_OUT), lambda i: (i, 0)),
        out_shape=jax.ShapeDtypeStruct((NP, D_OUT), jnp.float32),
    )(gx, wt, mask, w2pad)


def kernel(points, features, neighbor_indices, kernel_points, weights):
    px = points[:, 0]
    py = points[:, 1]
    pz = points[:, 2]
    idx_pad = jnp.pad(neighbor_indices, ((0, NP - N), (0, 0)))
    idx_flat = idx_pad.reshape(EP)
    kp_rep = jnp.tile(kernel_points.reshape(K * 3, 1), (1, 16)).reshape(720)
    w2pad = jnp.concatenate(
        [weights.reshape(K * D_IN, D_OUT),
         jnp.zeros((D_IN, D_OUT), jnp.float32)]).astype(jnp.bfloat16)
    mask = (jnp.arange(GP * 16)[:, None] // 16
            == jnp.arange(GE)[None, :] // H).astype(jnp.bfloat16)
    gx, wt = _sc_fused(features, px, py, pz, idx_flat, kp_rep)
    out = _tc_compute(gx, wt, mask, w2pad)
    return out[:N]


# speedup vs baseline: 2.7198x; 1.0748x over previous
import functools

import jax
import jax.numpy as jnp
from jax import lax
from jax.experimental import pallas as pl
from jax.experimental.pallas import tpu as pltpu
from jax.experimental.pallas import tpu_sc as plsc

N = 10000
H = 32
D_IN = 128
D_OUT = 128
K = 15
KP_EXTENT = 0.12

NP = 10240
EP = NP * H
NW = 32
EW = EP // NW
CHU = 256
NCH = EW // CHU
B = 256
GRID = NP // B

_MESH = dict(core_axis_name="c", subcore_axis_name="s")


def _sc_fused(feats, px, py, pz, idx_part, kp_rep, eoff, ne):
    mesh = plsc.VectorSubcoreMesh(**_MESH)
    ewl = ne // NW
    nchl = ewl // CHU

    @functools.partial(
        pl.kernel,
        out_type=(
            jax.ShapeDtypeStruct((ne, D_IN), jnp.float32),
            jax.ShapeDtypeStruct((16, ne), jnp.float32),
        ),
        mesh=mesh,
        compiler_params=pltpu.CompilerParams(needs_layout_passes=False),
        scratch_types=[
            pltpu.VMEM((N,), jnp.float32),
            pltpu.VMEM((N,), jnp.float32),
            pltpu.VMEM((N,), jnp.float32),
            pltpu.VMEM((ewl,), jnp.int32),
            pltpu.VMEM((720,), jnp.float32),
            pltpu.VMEM((16, CHU), jnp.float32),
            pltpu.VMEM((CHU, D_IN), jnp.float32),
            pltpu.VMEM((CHU, D_IN), jnp.float32),
            pltpu.VMEM((CHU,), jnp.int32),
            pltpu.VMEM((CHU,), jnp.int32),
            pltpu.SemaphoreType.DMA,
            pltpu.SemaphoreType.DMA,
            pltpu.SemaphoreType.DMA,
            pltpu.SemaphoreType.DMA,
        ],
    )
    def fused(feat_h, px_h, py_h, pz_h, idx_h, kp_h, gx_h, w_h,
              px_v, py_v, pz_v, idx_v, kp_v, wout_v,
              rows0, rows1, idxc0, idxc1,
              sin0, sin1, sout0, sout1):
        wid = lax.axis_index("s") * 2 + lax.axis_index("c")
        base = wid * ewl
        pltpu.sync_copy(px_h, px_v)
        pltpu.sync_copy(py_h, py_v)
        pltpu.sync_copy(pz_h, pz_v)
        pltpu.sync_copy(idx_h.at[pl.ds(base, ewl)], idx_v)
        pltpu.sync_copy(kp_h, kp_v)

        kvecs = [
            (kp_v[pl.ds(k * 48, 16)],
             kp_v[pl.ds(k * 48 + 16, 16)],
             kp_v[pl.ds(k * 48 + 32, 16)])
            for k in range(K)
        ]
        lane = lax.iota(jnp.int32, 16)
        inv_ext = jnp.float32(1.0 / KP_EXTENT)
        rows = (rows0, rows1)
        idxc = (idxc0, idxc1)
        sin = (sin0, sin1)
        sout = (sout0, sout1)

        def weights_for(c0):
            @pl.loop(0, CHU, step=16)
            def _vec(v0):
                i_nbr = idx_v[pl.ds(c0 + v0, 16)]
                e_g = eoff + base + c0 + v0 + lane
                i_ctr = lax.shift_right_logical(e_g, 5)
                xn = plsc.load_gather(px_v, [i_nbr])
                yn = plsc.load_gather(py_v, [i_nbr])
                zn = plsc.load_gather(pz_v, [i_nbr])
                xc = plsc.load_gather(px_v, [i_ctr])
                yc = plsc.load_gather(py_v, [i_ctr])
                zc = plsc.load_gather(pz_v, [i_ctr])
                rx = xn - xc
                ry = yn - yc
                rz = zn - zc
                row = v0 + lane
                for k in range(K):
                    kx, ky, kz = kvecs[k]
                    dx = rx - kx
                    dy = ry - ky
                    dz = rz - kz
                    d2 = jnp.maximum(dx * dx + dy * dy + dz * dz,
                                     jnp.float32(1e-24))
                    bits = plsc.bitcast(d2, jnp.int32)
                    seed = jnp.int32(0x5F3759DF) - lax.shift_right_logical(
                        bits, 1)
                    r = plsc.bitcast(seed, jnp.float32)
                    for _ in range(3):
                        r = r * (jnp.float32(1.5)
                                 - jnp.float32(0.5) * d2 * r * r)
                    dist = d2 * r
                    w = jnp.maximum(jnp.float32(1.0) - dist * inv_ext,
                                    jnp.float32(0.0))
                    kfull = jnp.full((16,), k, jnp.int32)
                    plsc.store_scatter(wout_v, [kfull, row], w)

            pltpu.sync_copy(wout_v, w_h.at[:, pl.ds(base + c0, CHU)])

        @pl.loop(0, nchl, step=2)
        def _pair(g):
            for b in range(2):
                gg = g + b
                c0 = gg * CHU

                @pl.when(g >= 2)
                def _drain():
                    pltpu.make_async_copy(
                        rows[b], gx_h.at[pl.ds(0, CHU), :],
                        sout[b]).wait()

                pltpu.sync_copy(idx_h.at[pl.ds(base + c0, CHU)], idxc[b])
                in_h = pltpu.async_copy(feat_h.at[idxc[b]], rows[b],
                                        sin[b])
                weights_for(c0)
                in_h.wait()
                pltpu.async_copy(rows[b],
                                 gx_h.at[pl.ds(base + c0, CHU), :],
                                 sout[b])

        for b in range(2):
            pltpu.make_async_copy(
                rows[b], gx_h.at[pl.ds(0, CHU), :], sout[b]).wait()

    return fused(feats, px, py, pz, idx_part, kp_rep)


GP = 32
GE = GP * H
NG = B // GP


def _tc_body(gx_ref, wt_ref, mask_ref, w2_ref, out_ref):
    feats = gx_ref[...].astype(jnp.bfloat16)
    mask = mask_ref[...]
    wfs = []
    for g in range(NG):
        wt_g = wt_ref[:, g * GE:(g + 1) * GE].astype(jnp.bfloat16)
        lhs = jnp.tile(wt_g, (GP, 1)) * mask
        x_g = feats[g * GE:(g + 1) * GE, :]
        acc = jnp.dot(lhs, x_g, preferred_element_type=jnp.float32)
        wfs.append(acc.reshape(GP, 16 * D_IN))
    wf = jnp.concatenate(wfs, axis=0).astype(jnp.bfloat16)
    out = jnp.dot(wf, w2_ref[...], preferred_element_type=jnp.float32)
    out_ref[...] = jnp.where(out > 0, out, 0.1 * out)


def _tc_compute(gx, wt, mask, w2pad, npts):
    return pl.pallas_call(
        _tc_body,
        grid=(npts // B,),
        in_specs=[
            pl.BlockSpec((B * H, D_IN), lambda i: (i, 0)),
            pl.BlockSpec((16, B * H), lambda i: (0, i)),
            pl.BlockSpec((GP * 16, GE), lambda i: (0, 0)),
            pl.BlockSpec((16 * D_IN, D_OUT), lambda i: (0, 0)),
        ],
        out_specs=pl.BlockSpec((B, D_OUT), lambda i: (i, 0)),
        out_shape=jax.ShapeDtypeStruct((npts, D_OUT), jnp.float32),
    )(gx, wt, mask, w2pad)


def kernel(points, features, neighbor_indices, kernel_points, weights):
    px = points[:, 0]
    py = points[:, 1]
    pz = points[:, 2]
    idx_pad = jnp.pad(neighbor_indices, ((0, NP - N), (0, 0)))
    idx_flat = idx_pad.reshape(EP)
    kp_rep = jnp.tile(kernel_points.reshape(K * 3, 1), (1, 16)).reshape(720)
    w2pad = jnp.concatenate(
        [weights.reshape(K * D_IN, D_OUT),
         jnp.zeros((D_IN, D_OUT), jnp.float32)]).astype(jnp.bfloat16)
    mask = (jnp.arange(GP * 16)[:, None] // 16
            == jnp.arange(GE)[None, :] // H).astype(jnp.bfloat16)
    eph = EP // 2
    nph = NP // 2
    outs = []
    for h in range(2):
        gxh, wth = _sc_fused(features, px, py, pz,
                             idx_flat[h * eph:(h + 1) * eph], kp_rep,
                             h * eph, eph)
        outs.append(_tc_compute(gxh, wth, mask, w2pad, nph))
    out = jnp.concatenate(outs, axis=0)
    return out[:N]


# speedup vs baseline: 2.7468x; 1.0099x over previous
import functools

import jax
import jax.numpy as jnp
from jax import lax
from jax.experimental import pallas as pl
from jax.experimental.pallas import tpu as pltpu
from jax.experimental.pallas import tpu_sc as plsc

N = 10000
H = 32
D_IN = 128
D_OUT = 128
K = 15
KP_EXTENT = 0.12

NP = 10240
EP = NP * H
NW = 32
EW = EP // NW
CHU = 256
NCH = EW // CHU
B = 256
GRID = NP // B

_MESH = dict(core_axis_name="c", subcore_axis_name="s")


def _sc_fused(feats, px, py, pz, idx_part, kp_rep, eoff, ne):
    mesh = plsc.VectorSubcoreMesh(**_MESH)
    ewl = ne // NW
    nchl = ewl // CHU

    @functools.partial(
        pl.kernel,
        out_type=(
            jax.ShapeDtypeStruct((ne, D_IN), jnp.float32),
            jax.ShapeDtypeStruct((16, ne), jnp.float32),
        ),
        mesh=mesh,
        compiler_params=pltpu.CompilerParams(needs_layout_passes=False),
        scratch_types=[
            pltpu.VMEM((N,), jnp.float32),
            pltpu.VMEM((N,), jnp.float32),
            pltpu.VMEM((N,), jnp.float32),
            pltpu.VMEM((ewl,), jnp.int32),
            pltpu.VMEM((720,), jnp.float32),
            pltpu.VMEM((16, CHU), jnp.float32),
            pltpu.VMEM((CHU, D_IN), jnp.float32),
            pltpu.VMEM((CHU, D_IN), jnp.float32),
            pltpu.VMEM((CHU,), jnp.int32),
            pltpu.VMEM((CHU,), jnp.int32),
            pltpu.SemaphoreType.DMA,
            pltpu.SemaphoreType.DMA,
            pltpu.SemaphoreType.DMA,
            pltpu.SemaphoreType.DMA,
        ],
    )
    def fused(feat_h, px_h, py_h, pz_h, idx_h, kp_h, gx_h, w_h,
              px_v, py_v, pz_v, idx_v, kp_v, wout_v,
              rows0, rows1, idxc0, idxc1,
              sin0, sin1, sout0, sout1):
        wid = lax.axis_index("s") * 2 + lax.axis_index("c")
        base = wid * ewl
        pltpu.sync_copy(px_h, px_v)
        pltpu.sync_copy(py_h, py_v)
        pltpu.sync_copy(pz_h, pz_v)
        pltpu.sync_copy(idx_h.at[pl.ds(base, ewl)], idx_v)
        pltpu.sync_copy(kp_h, kp_v)

        kvecs = [
            (kp_v[pl.ds(k * 48, 16)],
             kp_v[pl.ds(k * 48 + 16, 16)],
             kp_v[pl.ds(k * 48 + 32, 16)])
            for k in range(K)
        ]
        lane = lax.iota(jnp.int32, 16)
        inv_ext = jnp.float32(1.0 / KP_EXTENT)
        rows = (rows0, rows1)
        idxc = (idxc0, idxc1)
        sin = (sin0, sin1)
        sout = (sout0, sout1)

        def weights_for(c0):
            @pl.loop(0, CHU, step=16)
            def _vec(v0):
                i_nbr = idx_v[pl.ds(c0 + v0, 16)]
                e_g = eoff + base + c0 + v0 + lane
                i_ctr = lax.shift_right_logical(e_g, 5)
                xn = plsc.load_gather(px_v, [i_nbr])
                yn = plsc.load_gather(py_v, [i_nbr])
                zn = plsc.load_gather(pz_v, [i_nbr])
                xc = plsc.load_gather(px_v, [i_ctr])
                yc = plsc.load_gather(py_v, [i_ctr])
                zc = plsc.load_gather(pz_v, [i_ctr])
                rx = xn - xc
                ry = yn - yc
                rz = zn - zc
                row = v0 + lane
                for k in range(K):
                    kx, ky, kz = kvecs[k]
                    dx = rx - kx
                    dy = ry - ky
                    dz = rz - kz
                    d2 = jnp.maximum(dx * dx + dy * dy + dz * dz,
                                     jnp.float32(1e-24))
                    bits = plsc.bitcast(d2, jnp.int32)
                    seed = jnp.int32(0x5F3759DF) - lax.shift_right_logical(
                        bits, 1)
                    r = plsc.bitcast(seed, jnp.float32)
                    for _ in range(3):
                        r = r * (jnp.float32(1.5)
                                 - jnp.float32(0.5) * d2 * r * r)
                    dist = d2 * r
                    w = jnp.maximum(jnp.float32(1.0) - dist * inv_ext,
                                    jnp.float32(0.0))
                    kfull = jnp.full((16,), k, jnp.int32)
                    plsc.store_scatter(wout_v, [kfull, row], w)

            pltpu.sync_copy(wout_v, w_h.at[:, pl.ds(base + c0, CHU)])

        @pl.loop(0, nchl, step=2)
        def _pair(g):
            for b in range(2):
                gg = g + b
                c0 = gg * CHU

                @pl.when(g >= 2)
                def _drain():
                    pltpu.make_async_copy(
                        rows[b], gx_h.at[pl.ds(0, CHU), :],
                        sout[b]).wait()

                pltpu.sync_copy(idx_h.at[pl.ds(base + c0, CHU)], idxc[b])
                in_h = pltpu.async_copy(feat_h.at[idxc[b]], rows[b],
                                        sin[b])
                weights_for(c0)
                in_h.wait()
                pltpu.async_copy(rows[b],
                                 gx_h.at[pl.ds(base + c0, CHU), :],
                                 sout[b])

        for b in range(2):
            pltpu.make_async_copy(
                rows[b], gx_h.at[pl.ds(0, CHU), :], sout[b]).wait()

    return fused(feats, px, py, pz, idx_part, kp_rep)


GP = 32
GE = GP * H
NG = B // GP


def _tc_body(gx_ref, wt_ref, mask_ref, w2_ref, out_ref):
    feats = gx_ref[...].astype(jnp.bfloat16)
    mask = mask_ref[...]
    wfs = []
    for g in range(NG):
        wt_g = wt_ref[:, g * GE:(g + 1) * GE].astype(jnp.bfloat16)
        lhs = jnp.tile(wt_g, (GP, 1)) * mask
        x_g = feats[g * GE:(g + 1) * GE, :]
        acc = jnp.dot(lhs, x_g, preferred_element_type=jnp.float32)
        wfs.append(acc.reshape(GP, 16 * D_IN))
    wf = jnp.concatenate(wfs, axis=0).astype(jnp.bfloat16)
    out = jnp.dot(wf, w2_ref[...], preferred_element_type=jnp.float32)
    out_ref[...] = jnp.where(out > 0, out, 0.1 * out)


def _tc_compute(gx, wt, mask, w2pad, npts):
    return pl.pallas_call(
        _tc_body,
        grid=(npts // B,),
        in_specs=[
            pl.BlockSpec((B * H, D_IN), lambda i: (i, 0)),
            pl.BlockSpec((16, B * H), lambda i: (0, i)),
            pl.BlockSpec((GP * 16, GE), lambda i: (0, 0)),
            pl.BlockSpec((16 * D_IN, D_OUT), lambda i: (0, 0)),
        ],
        out_specs=pl.BlockSpec((B, D_OUT), lambda i: (i, 0)),
        out_shape=jax.ShapeDtypeStruct((npts, D_OUT), jnp.float32),
    )(gx, wt, mask, w2pad)


def kernel(points, features, neighbor_indices, kernel_points, weights):
    px = points[:, 0]
    py = points[:, 1]
    pz = points[:, 2]
    idx_pad = jnp.pad(neighbor_indices, ((0, NP - N), (0, 0)))
    idx_flat = idx_pad.reshape(EP)
    kp_rep = jnp.tile(kernel_points.reshape(K * 3, 1), (1, 16)).reshape(720)
    w2pad = jnp.concatenate(
        [weights.reshape(K * D_IN, D_OUT),
         jnp.zeros((D_IN, D_OUT), jnp.float32)]).astype(jnp.bfloat16)
    mask = (jnp.arange(GP * 16)[:, None] // 16
            == jnp.arange(GE)[None, :] // H).astype(jnp.bfloat16)
    eph = EP // 4
    nph = NP // 4
    outs = []
    for h in range(4):
        gxh, wth = _sc_fused(features, px, py, pz,
                             idx_flat[h * eph:(h + 1) * eph], kp_rep,
                             h * eph, eph)
        outs.append(_tc_compute(gxh, wth, mask, w2pad, nph))
    out = jnp.concatenate(outs, axis=0)
    return out[:N]


# speedup vs baseline: 2.7511x; 1.0016x over previous
import functools

import jax
import jax.numpy as jnp
from jax import lax
from jax.experimental import pallas as pl
from jax.experimental.pallas import tpu as pltpu
from jax.experimental.pallas import tpu_sc as plsc

N = 10000
H = 32
D_IN = 128
D_OUT = 128
K = 15
KP_EXTENT = 0.12

NP = 10240
EP = NP * H
NW = 32
CHU = 256
B = 256

_MESH = dict(core_axis_name="c", subcore_axis_name="s")


def _sc_fused(feats, px, py, pz, idx_part, kp_rep, eoff, ne):
    mesh = plsc.VectorSubcoreMesh(**_MESH)
    ewl = ne // NW
    nchl = ewl // CHU

    @functools.partial(
        pl.kernel,
        out_type=(
            jax.ShapeDtypeStruct((ne, D_IN), jnp.float32),
            jax.ShapeDtypeStruct((16, ne), jnp.float32),
        ),
        mesh=mesh,
        compiler_params=pltpu.CompilerParams(needs_layout_passes=False),
        scratch_types=[
            pltpu.VMEM((N,), jnp.float32),
            pltpu.VMEM((N,), jnp.float32),
            pltpu.VMEM((N,), jnp.float32),
            pltpu.VMEM((ewl,), jnp.int32),
            pltpu.VMEM((720,), jnp.float32),
            pltpu.VMEM((16, CHU), jnp.float32),
            pltpu.VMEM((CHU, D_IN), jnp.float32),
            pltpu.VMEM((CHU, D_IN), jnp.float32),
            pltpu.VMEM((CHU,), jnp.int32),
            pltpu.VMEM((CHU,), jnp.int32),
            pltpu.SemaphoreType.DMA,
            pltpu.SemaphoreType.DMA,
            pltpu.SemaphoreType.DMA,
            pltpu.SemaphoreType.DMA,
        ],
    )
    def fused(feat_h, px_h, py_h, pz_h, idx_h, kp_h, gx_h, w_h,
              px_v, py_v, pz_v, idx_v, kp_v, wout_v,
              rows0, rows1, idxc0, idxc1,
              sin0, sin1, sout0, sout1):
        wid = lax.axis_index("s") * 2 + lax.axis_index("c")
        base = wid * ewl
        pltpu.sync_copy(px_h, px_v)
        pltpu.sync_copy(py_h, py_v)
        pltpu.sync_copy(pz_h, pz_v)
        pltpu.sync_copy(idx_h.at[pl.ds(base, ewl)], idx_v)
        pltpu.sync_copy(kp_h, kp_v)

        kvecs = [
            (kp_v[pl.ds(k * 48, 16)],
             kp_v[pl.ds(k * 48 + 16, 16)],
             kp_v[pl.ds(k * 48 + 32, 16)])
            for k in range(K)
        ]
        lane = lax.iota(jnp.int32, 16)
        inv_ext = jnp.float32(1.0 / KP_EXTENT)
        rows = (rows0, rows1)
        idxc = (idxc0, idxc1)
        sin = (sin0, sin1)
        sout = (sout0, sout1)

        def weights_for(c0):
            @pl.loop(0, CHU, step=16)
            def _vec(v0):
                i_nbr = idx_v[pl.ds(c0 + v0, 16)]
                e_g = eoff + base + c0 + v0 + lane
                i_ctr = lax.shift_right_logical(e_g, 5)
                xn = plsc.load_gather(px_v, [i_nbr])
                yn = plsc.load_gather(py_v, [i_nbr])
                zn = plsc.load_gather(pz_v, [i_nbr])
                xc = plsc.load_gather(px_v, [i_ctr])
                yc = plsc.load_gather(py_v, [i_ctr])
                zc = plsc.load_gather(pz_v, [i_ctr])
                rx = xn - xc
                ry = yn - yc
                rz = zn - zc
                row = v0 + lane
                for k in range(K):
                    kx, ky, kz = kvecs[k]
                    dx = rx - kx
                    dy = ry - ky
                    dz = rz - kz
                    d2 = jnp.maximum(dx * dx + dy * dy + dz * dz,
                                     jnp.float32(1e-24))
                    bits = plsc.bitcast(d2, jnp.int32)
                    seed = jnp.int32(0x5F3759DF) - lax.shift_right_logical(
                        bits, 1)
                    r = plsc.bitcast(seed, jnp.float32)
                    for _ in range(3):
                        r = r * (jnp.float32(1.5)
                                 - jnp.float32(0.5) * d2 * r * r)
                    dist = d2 * r
                    w = jnp.maximum(jnp.float32(1.0) - dist * inv_ext,
                                    jnp.float32(0.0))
                    kfull = jnp.full((16,), k, jnp.int32)
                    plsc.store_scatter(wout_v, [kfull, row], w)

            pltpu.sync_copy(wout_v, w_h.at[:, pl.ds(base + c0, CHU)])

        @pl.loop(0, nchl, step=2)
        def _pair(g):
            for b in range(2):
                gg = g + b
                c0 = gg * CHU

                @pl.when(g >= 2)
                def _drain():
                    pltpu.make_async_copy(
                        rows[b], gx_h.at[pl.ds(0, CHU), :],
                        sout[b]).wait()

                pltpu.sync_copy(idx_h.at[pl.ds(base + c0, CHU)], idxc[b])
                in_h = pltpu.async_copy(feat_h.at[idxc[b]], rows[b],
                                        sin[b])
                weights_for(c0)
                in_h.wait()
                pltpu.async_copy(rows[b],
                                 gx_h.at[pl.ds(base + c0, CHU), :],
                                 sout[b])

        for b in range(2):
            pltpu.make_async_copy(
                rows[b], gx_h.at[pl.ds(0, CHU), :], sout[b]).wait()

    return fused(feats, px, py, pz, idx_part, kp_rep)


GP = 32
GE = GP * H
NG = B // GP


def _tc_body(gx_ref, wt_ref, mask_ref, w2_ref, out_ref):
    feats = gx_ref[...].astype(jnp.bfloat16)
    mask = mask_ref[...]
    wfs = []
    for g in range(NG):
        wt_g = wt_ref[:, g * GE:(g + 1) * GE].astype(jnp.bfloat16)
        lhs = jnp.tile(wt_g, (GP, 1)) * mask
        x_g = feats[g * GE:(g + 1) * GE, :]
        acc = jnp.dot(lhs, x_g, preferred_element_type=jnp.float32)
        wfs.append(acc.reshape(GP, 16 * D_IN))
    wf = jnp.concatenate(wfs, axis=0).astype(jnp.bfloat16)
    out = jnp.dot(wf, w2_ref[...], preferred_element_type=jnp.float32)
    out_ref[...] = jnp.where(out > 0, out, 0.1 * out)


def _tc_compute(gx, wt, mask, w2pad, npts):
    return pl.pallas_call(
        _tc_body,
        grid=(npts // B,),
        in_specs=[
            pl.BlockSpec((B * H, D_IN), lambda i: (i, 0)),
            pl.BlockSpec((16, B * H), lambda i: (0, i)),
            pl.BlockSpec((GP * 16, GE), lambda i: (0, 0)),
            pl.BlockSpec((16 * D_IN, D_OUT), lambda i: (0, 0)),
        ],
        out_specs=pl.BlockSpec((B, D_OUT), lambda i: (i, 0)),
        out_shape=jax.ShapeDtypeStruct((npts, D_OUT), jnp.float32),
    )(gx, wt, mask, w2pad)


def kernel(points, features, neighbor_indices, kernel_points, weights):
    px = points[:, 0]
    py = points[:, 1]
    pz = points[:, 2]
    idx_pad = jnp.pad(neighbor_indices, ((0, NP - N), (0, 0)))
    idx_flat = idx_pad.reshape(EP)
    kp_rep = jnp.tile(kernel_points.reshape(K * 3, 1), (1, 16)).reshape(720)
    w2pad = jnp.concatenate(
        [weights.reshape(K * D_IN, D_OUT),
         jnp.zeros((D_IN, D_OUT), jnp.float32)]).astype(jnp.bfloat16)
    mask = (jnp.arange(GP * 16)[:, None] // 16
            == jnp.arange(GE)[None, :] // H).astype(jnp.bfloat16)
    eph = EP // 4
    nph = NP // 4
    outs = []
    for h in range(4):
        gxh, wth = _sc_fused(features, px, py, pz,
                             idx_flat[h * eph:(h + 1) * eph], kp_rep,
                             h * eph, eph)
        outs.append(_tc_compute(gxh, wth, mask, w2pad, nph))
    out = jnp.concatenate(outs, axis=0)
    return out[:N]
